# baseline (speedup 1.0000x reference)
import jax
import jax.numpy as jnp
from jax.experimental import pallas as pl
from jax.experimental.pallas import tpu as pltpu

BB = 4


def _body(hour_ref, day_ref, ht_ref, dt_ref, x_ref, o_ref):
    pid = pl.program_id(0)
    T = hour_ref.shape[1]
    rows = []
    for i in range(BB):
        b = pid * BB + i
        parts = []
        for t in range(T):
            h = hour_ref[b, t]
            d = day_ref[b, t]
            row = jnp.concatenate(
                [ht_ref[pl.ds(h, 1), :], dt_ref[pl.ds(d, 1), :]], axis=1
            )
            parts.append(row)
        rows.append(jnp.concatenate(parts, axis=0)[None, None])
    emb = jnp.concatenate(rows, axis=0)
    o_ref[...] = x_ref[...] + emb


def kernel(x, hour, day, hour_table, day_table):
    B, N, T, D = x.shape
    hour = hour.astype(jnp.int32)
    day = day.astype(jnp.int32)

    grid_spec = pltpu.PrefetchScalarGridSpec(
        num_scalar_prefetch=2,
        grid=(B // BB,),
        in_specs=[
            pl.BlockSpec(hour_table.shape, lambda b, *_: (0, 0)),
            pl.BlockSpec(day_table.shape, lambda b, *_: (0, 0)),
            pl.BlockSpec((BB, N, T, D), lambda b, *_: (b, 0, 0, 0)),
        ],
        out_specs=pl.BlockSpec((BB, N, T, D), lambda b, *_: (b, 0, 0, 0)),
    )
    return pl.pallas_call(
        _body,
        grid_spec=grid_spec,
        out_shape=jax.ShapeDtypeStruct((B, N, T, D), x.dtype),
    )(hour, day, hour_table, day_table, x)
